# baseline (speedup 1.0000x reference)
import functools

import jax
import jax.numpy as jnp
from jax import lax
from jax.experimental import pallas as pl
from jax.experimental.pallas import tpu as pltpu
from jax.experimental.pallas import tpu_sc as plsc

BATCH = 4096
SEQ = 200
EMBED_DIM = 64
NUM_IDX = BATCH * SEQ

NUM_CORES = 2
NUM_SUBCORES = 16
NUM_WORKERS = NUM_CORES * NUM_SUBCORES
PER_WORKER = NUM_IDX // NUM_WORKERS
CHUNK = 128
NUM_CHUNKS = PER_WORKER // CHUNK

_mesh = plsc.VectorSubcoreMesh(core_axis_name="c", subcore_axis_name="s")


@functools.partial(
    pl.kernel,
    mesh=_mesh,
    out_type=jax.ShapeDtypeStruct((NUM_WORKERS, PER_WORKER, EMBED_DIM), jnp.float32),
    scratch_types=[
        pltpu.VMEM((NUM_CHUNKS, CHUNK), jnp.int32),
        pltpu.VMEM((CHUNK, EMBED_DIM), jnp.float32),
        pltpu.SemaphoreType.DMA,
    ],
    compiler_params=pltpu.CompilerParams(use_tc_tiling_on_sc=False),
)
def _embed_sc(x_hbm, table_hbm, out_hbm, idx_v, rows_v, sem):
    wid = lax.axis_index("s") * NUM_CORES + lax.axis_index("c")
    pltpu.sync_copy(x_hbm.at[wid], idx_v)

    def body(j, carry):
        pltpu.async_copy(table_hbm.at[idx_v.at[j]], rows_v, sem).wait()
        pltpu.sync_copy(rows_v, out_hbm.at[wid, pl.ds(j * CHUNK, CHUNK)])
        return carry

    lax.fori_loop(0, NUM_CHUNKS, body, 0)


def kernel(x, table):
    xr = x.reshape(NUM_WORKERS, NUM_CHUNKS, CHUNK)
    out = _embed_sc(xr, table)
    return out.reshape(BATCH, SEQ, EMBED_DIM)


# speedup vs baseline: 1.1147x; 1.1147x over previous
import functools

import jax
import jax.numpy as jnp
from jax import lax
from jax.experimental import pallas as pl
from jax.experimental.pallas import tpu as pltpu
from jax.experimental.pallas import tpu_sc as plsc

BATCH = 4096
SEQ = 200
EMBED_DIM = 64
NUM_IDX = BATCH * SEQ

NUM_CORES = 2
NUM_SUBCORES = 16
NUM_WORKERS = NUM_CORES * NUM_SUBCORES
PER_WORKER = NUM_IDX // NUM_WORKERS
CHUNK = 128
NUM_CHUNKS = PER_WORKER // CHUNK
NBUF = 8
OUTER = NUM_CHUNKS // NBUF

_mesh = plsc.VectorSubcoreMesh(core_axis_name="c", subcore_axis_name="s")


@functools.partial(
    pl.kernel,
    mesh=_mesh,
    out_type=jax.ShapeDtypeStruct((NUM_WORKERS, PER_WORKER, EMBED_DIM), jnp.float32),
    scratch_types=[
        pltpu.VMEM((NUM_CHUNKS, CHUNK), jnp.int32),
        pltpu.VMEM((NBUF, CHUNK, EMBED_DIM), jnp.float32),
    ]
    + [pltpu.SemaphoreType.DMA] * (2 * NBUF),
    compiler_params=pltpu.CompilerParams(use_tc_tiling_on_sc=False),
)
def _embed_sc(x_hbm, table_hbm, out_hbm, idx_v, rows_v, *sems):
    sem_g = sems[:NBUF]
    sem_w = sems[NBUF:]
    wid = lax.axis_index("s") * NUM_CORES + lax.axis_index("c")
    pltpu.sync_copy(x_hbm.at[wid], idx_v)

    for b in range(NBUF - 1):
        pltpu.async_copy(table_hbm.at[idx_v.at[b]], rows_v.at[b], sem_g[b])

    def outer(g, carry):
        for b in range(NBUF):
            j = g * NBUF + b
            pltpu.make_async_copy(
                table_hbm.at[idx_v.at[j]], rows_v.at[b], sem_g[b]
            ).wait()
            pltpu.async_copy(
                rows_v.at[b],
                out_hbm.at[wid, pl.ds(j * CHUNK, CHUNK)],
                sem_w[b],
            )
            jn = j + NBUF - 1
            bp = (b + NBUF - 1) % NBUF

            @pl.when(jn < NUM_CHUNKS)
            def _():
                @pl.when(j >= 1)
                def _():
                    pltpu.make_async_copy(
                        rows_v.at[bp],
                        out_hbm.at[wid, pl.ds(0, CHUNK)],
                        sem_w[bp],
                    ).wait()

                pltpu.async_copy(table_hbm.at[idx_v.at[jn]], rows_v.at[bp], sem_g[bp])

        return carry

    lax.fori_loop(0, OUTER, outer, 0)

    for b in range(NBUF):
        pltpu.make_async_copy(
            rows_v.at[b], out_hbm.at[wid, pl.ds(0, CHUNK)], sem_w[b]
        ).wait()


def kernel(x, table):
    xr = x.reshape(NUM_WORKERS, NUM_CHUNKS, CHUNK)
    out = _embed_sc(xr, table)
    return out.reshape(BATCH, SEQ, EMBED_DIM)


# speedup vs baseline: 1.1202x; 1.0049x over previous
import functools

import jax
import jax.numpy as jnp
from jax import lax
from jax.experimental import pallas as pl
from jax.experimental.pallas import tpu as pltpu
from jax.experimental.pallas import tpu_sc as plsc

BATCH = 4096
SEQ = 200
EMBED_DIM = 64

NUM_CORES = 2
NUM_SUBCORES = 16
NUM_WORKERS = NUM_CORES * NUM_SUBCORES
BLOCK_B = BATCH // NUM_WORKERS
NBUF = 8

_mesh = plsc.VectorSubcoreMesh(core_axis_name="c", subcore_axis_name="s")


@functools.partial(
    pl.kernel,
    mesh=_mesh,
    out_type=jax.ShapeDtypeStruct((BATCH, SEQ, EMBED_DIM), jnp.float32),
    scratch_types=[
        pltpu.VMEM((SEQ, BLOCK_B), jnp.int32),
        pltpu.VMEM((NBUF, BLOCK_B, EMBED_DIM), jnp.float32),
    ]
    + [pltpu.SemaphoreType.DMA] * (2 * NBUF),
    compiler_params=pltpu.CompilerParams(use_tc_tiling_on_sc=False),
)
def _embed_sc(xt_hbm, table_hbm, out_hbm, idx_v, rows_v, *sems):
    sem_g = sems[:NBUF]
    sem_w = sems[NBUF:]
    wid = lax.axis_index("s") * NUM_CORES + lax.axis_index("c")
    b0 = wid * BLOCK_B
    pltpu.sync_copy(xt_hbm.at[:, pl.ds(b0, BLOCK_B)], idx_v)

    for b in range(NBUF - 1):
        pltpu.async_copy(table_hbm.at[idx_v.at[b]], rows_v.at[b], sem_g[b])

    def outer(g, carry):
        for b in range(NBUF):
            s = g * NBUF + b
            pltpu.make_async_copy(
                table_hbm.at[idx_v.at[s]], rows_v.at[b], sem_g[b]
            ).wait()
            pltpu.async_copy(
                rows_v.at[b],
                out_hbm.at[pl.ds(b0, BLOCK_B), s],
                sem_w[b],
            )
            sn = s + NBUF - 1
            bp = (b + NBUF - 1) % NBUF

            @pl.when(sn < SEQ)
            def _():
                @pl.when(s >= 1)
                def _():
                    pltpu.make_async_copy(
                        rows_v.at[bp],
                        out_hbm.at[pl.ds(b0, BLOCK_B), 0],
                        sem_w[bp],
                    ).wait()

                pltpu.async_copy(table_hbm.at[idx_v.at[sn]], rows_v.at[bp], sem_g[bp])

        return carry

    lax.fori_loop(0, SEQ // NBUF, outer, 0)

    for b in range(NBUF):
        pltpu.make_async_copy(
            rows_v.at[b], out_hbm.at[pl.ds(b0, BLOCK_B), 0], sem_w[b]
        ).wait()


def kernel(x, table):
    return _embed_sc(x.T, table)
